# baseline (speedup 1.0000x reference)
import functools

import jax
import jax.numpy as jnp
from jax import lax
from jax.experimental import pallas as pl
from jax.experimental.pallas import tpu as pltpu
from jax.experimental.pallas import tpu_sc as plsc

BATCH = 4096
HIST_LEN = 200
EMBED_DIM = 32

NUM_CORES = 2
NUM_SUBCORES = 16
NUM_WORKERS = NUM_CORES * NUM_SUBCORES

CHUNK = 512
CB = CHUNK // 128
CPT = BATCH // CHUNK
NUM_UNITS = HIST_LEN * CPT
UPW = NUM_UNITS // NUM_WORKERS
TBLK = 4 * CB * 8 * 128


@functools.partial(
    pl.kernel,
    mesh=plsc.VectorSubcoreMesh(core_axis_name="c", subcore_axis_name="s"),
    out_type=jax.ShapeDtypeStruct((HIST_LEN, 4, 32 * 8 * 128), jnp.float32),
    scratch_types=[
        pltpu.VMEM((2, CHUNK), jnp.int32),
        pltpu.VMEM((2, CHUNK, EMBED_DIM), jnp.float32),
        pltpu.VMEM((2, TBLK), jnp.float32),
        pltpu.SemaphoreType.DMA((2,)),
        pltpu.SemaphoreType.DMA((2,)),
    ],
    compiler_params=pltpu.CompilerParams(use_tc_tiling_on_sc=False,
                                         needs_layout_passes=False),
)
def _embed(x_hbm, table_hbm, out_hbm, idx_v, rows_v, trans_v, gsem, ssem):
    wid = lax.axis_index("s") * NUM_CORES + lax.axis_index("c")
    u0 = wid * UPW

    iota = lax.iota(jnp.int32, 16)
    cvec = [((dl * 2 + (iota >> 3)) * (CB * 1024) + (iota & 7) * 128)
            for dl in range(2)]

    def fire_gather(u, p):
        t = u // CPT
        c = lax.rem(u, CPT)
        pltpu.sync_copy(x_hbm.at[t, c], idx_v.at[p])
        pltpu.async_copy(table_hbm.at[idx_v.at[p]], rows_v.at[p], gsem.at[p])

    def wait_gather(p):
        pltpu.make_async_copy(table_hbm.at[idx_v.at[p]], rows_v.at[p],
                              gsem.at[p]).wait()

    def store_descs(u, p):
        t = u // CPT
        c = lax.rem(u, CPT)
        return [
            pltpu.make_async_copy(
                trans_v.at[p, pl.ds(dt * CB * 1024, CB * 1024)],
                out_hbm.at[t, dt, pl.ds(c * CB * 1024, CB * 1024)],
                ssem.at[p],
            )
            for dt in range(4)
        ]

    def transpose(p):
        @pl.loop(0, CHUNK, unroll=8)
        def _(r):
            s = (r >> 7) * (8 * 128) + (r & 127)
            for dl in range(2):
                v = rows_v[p, r, pl.ds(dl * 16, 16)]
                plsc.store_scatter(trans_v.at[p], [cvec[dl] + s], v)

    fire_gather(u0, 0)
    fire_gather(u0 + 1, 1)

    @pl.loop(0, UPW // 2)
    def _(g):
        for p in range(2):
            u = u0 + g * 2 + p
            wait_gather(p)

            @pl.when(g > 0)
            def _():
                for d in store_descs(u - 2, p):
                    d.wait()

            transpose(p)
            for d in store_descs(u, p):
                d.start()

            @pl.when(g < UPW // 2 - 1)
            def _():
                fire_gather(u + 2, p)

    for p in range(2):
        u_last = u0 + UPW - 2 + p
        for d in store_descs(u_last, p):
            d.wait()


@jax.jit
def kernel(x, table):
    xt = x.T.reshape(HIST_LEN, CPT, CHUNK).astype(jnp.int32)
    flat = _embed(xt, table)
    out5 = flat.reshape(HIST_LEN, 4, 32, 8, 128)
    return out5.transpose(2, 4, 0, 1, 3).reshape(BATCH, HIST_LEN, EMBED_DIM)


# speedup vs baseline: 1.1595x; 1.1595x over previous
import functools

import jax
import jax.numpy as jnp
from jax import lax
from jax.experimental import pallas as pl
from jax.experimental.pallas import tpu as pltpu
from jax.experimental.pallas import tpu_sc as plsc

BATCH = 4096
HIST_LEN = 200
EMBED_DIM = 32

NUM_CORES = 2
NUM_SUBCORES = 16
NUM_WORKERS = NUM_CORES * NUM_SUBCORES

CHUNK = 512
CB = CHUNK // 128
CPT = BATCH // CHUNK
NUM_UNITS = HIST_LEN * CPT
UPW = NUM_UNITS // NUM_WORKERS
TBLK = 4 * CB * 8 * 128


@functools.partial(
    pl.kernel,
    mesh=plsc.VectorSubcoreMesh(core_axis_name="c", subcore_axis_name="s"),
    out_type=jax.ShapeDtypeStruct((HIST_LEN, 4, 32 * 8 * 128), jnp.float32),
    scratch_types=[
        pltpu.VMEM((2, CHUNK), jnp.int32),
        pltpu.VMEM((2, CHUNK, EMBED_DIM), jnp.float32),
        pltpu.VMEM((2, TBLK), jnp.float32),
        pltpu.SemaphoreType.DMA((2,)),
        pltpu.SemaphoreType.DMA((2,)),
    ],
    compiler_params=pltpu.CompilerParams(use_tc_tiling_on_sc=False,
                                         needs_layout_passes=False),
)
def _embed(x_hbm, table_hbm, out_hbm, idx_v, rows_v, trans_v, gsem, ssem):
    wid = lax.axis_index("s") * NUM_CORES + lax.axis_index("c")
    u0 = wid * UPW

    iota = lax.iota(jnp.int32, 16)
    cvec = [((dl * 2 + (iota >> 3)) * (CB * 1024) + (iota & 7) * 128)
            for dl in range(2)]

    def fire_gather(u, p):
        t = u // CPT
        c = lax.rem(u, CPT)
        pltpu.sync_copy(x_hbm.at[t, c], idx_v.at[p])
        pltpu.async_copy(table_hbm.at[idx_v.at[p]], rows_v.at[p], gsem.at[p])

    def wait_gather(p):
        pltpu.make_async_copy(table_hbm.at[idx_v.at[p]], rows_v.at[p],
                              gsem.at[p]).wait()

    def store_descs(u, p):
        t = u // CPT
        c = lax.rem(u, CPT)
        return [
            pltpu.make_async_copy(
                trans_v.at[p, pl.ds(dt * CB * 1024, CB * 1024)],
                out_hbm.at[t, dt, pl.ds(c * CB * 1024, CB * 1024)],
                ssem.at[p],
            )
            for dt in range(4)
        ]

    def transpose(p):
        @plsc.parallel_loop(0, CHUNK, unroll=8)
        def _(r):
            s = (r >> 7) * (8 * 128) + (r & 127)
            for dl in range(2):
                v = rows_v[p, r, pl.ds(dl * 16, 16)]
                plsc.store_scatter(trans_v.at[p], [cvec[dl] + s], v)

    fire_gather(u0, 0)
    fire_gather(u0 + 1, 1)

    @pl.loop(0, UPW // 2)
    def _(g):
        for p in range(2):
            u = u0 + g * 2 + p
            wait_gather(p)

            @pl.when(g > 0)
            def _():
                for d in store_descs(u - 2, p):
                    d.wait()

            transpose(p)
            for d in store_descs(u, p):
                d.start()

            @pl.when(g < UPW // 2 - 1)
            def _():
                fire_gather(u + 2, p)

    for p in range(2):
        u_last = u0 + UPW - 2 + p
        for d in store_descs(u_last, p):
            d.wait()


@jax.jit
def kernel(x, table):
    xt = x.T.reshape(HIST_LEN, CPT, CHUNK).astype(jnp.int32)
    flat = _embed(xt, table)
    out5 = flat.reshape(HIST_LEN, 4, 32, 8, 128)
    return out5.transpose(2, 4, 0, 1, 3).reshape(BATCH, HIST_LEN, EMBED_DIM)
